# Initial kernel scaffold; baseline (speedup 1.0000x reference)
import jax
import jax.numpy as jnp
from jax.experimental import pallas as pl


def kernel(x, token_table, pos_table):
    raise NotImplementedError("write your pallas kernel here")


# speedup vs baseline: 4.4617x; 4.4617x over previous
import functools

import jax
import jax.numpy as jnp
from jax import lax
from jax.experimental import pallas as pl
from jax.experimental.pallas import tpu as pltpu
from jax.experimental.pallas import tpu_sc as plsc

MAXLEN = 200
EMBED = 32
BATCH = 16384

NC = 2
NS = 16
NW = NC * NS

SEQ_PER_W = BATCH // NW
C = 4
ROWS = C * MAXLEN
NCHUNK = SEQ_PER_W // C
GSUB = 100
NG = ROWS // GSUB
XROW_PER_W = (SEQ_PER_W * MAXLEN) // GSUB
ROW_PER_W = SEQ_PER_W * MAXLEN

_mesh = plsc.VectorSubcoreMesh(core_axis_name="c", subcore_axis_name="s")


@functools.partial(
    pl.kernel,
    mesh=_mesh,
    compiler_params=pltpu.CompilerParams(use_tc_tiling_on_sc=False),
    out_type=jax.ShapeDtypeStruct((BATCH * MAXLEN, EMBED), jnp.float32),
    scratch_types=[
        pltpu.VMEM((NG, GSUB), jnp.int32),
        pltpu.VMEM((ROWS, EMBED), jnp.float32),
        pltpu.VMEM((MAXLEN, EMBED), jnp.float32),
        pltpu.SemaphoreType.DMA,
    ],
)
def _embed_kernel(x_hbm, tok_hbm, pos_hbm, out_hbm, idx_v, buf, pos_v, sem):
    wid = lax.axis_index("s") * NC + lax.axis_index("c")
    pltpu.sync_copy(pos_hbm, pos_v)

    def chunk_body(g, carry):
        xb = wid * XROW_PER_W + g * NG
        rb = wid * ROW_PER_W + g * ROWS
        pltpu.sync_copy(x_hbm.at[pl.ds(xb, NG)], idx_v)
        copies = [
            pltpu.async_copy(
                tok_hbm.at[idx_v.at[j]], buf.at[pl.ds(j * GSUB, GSUB)], sem
            )
            for j in range(NG)
        ]
        for cp in copies:
            cp.wait()

        def add_body(t, c2):
            p0 = pos_v[t, pl.ds(0, 16)]
            p1 = pos_v[t, pl.ds(16, 16)]
            for s in range(C):
                r = s * MAXLEN + t
                buf[r, pl.ds(0, 16)] += p0
                buf[r, pl.ds(16, 16)] += p1
            return c2

        lax.fori_loop(0, MAXLEN, add_body, 0)
        pltpu.sync_copy(buf, out_hbm.at[pl.ds(rb, ROWS)])
        return carry

    lax.fori_loop(0, NCHUNK, chunk_body, 0)


def kernel(x, token_table, pos_table):
    x_flat = x.reshape(-1).astype(jnp.int32).reshape(-1, GSUB)
    out = _embed_kernel(x_flat, token_table, pos_table)
    return out.reshape(BATCH, MAXLEN, EMBED)


# speedup vs baseline: 4.9927x; 1.1190x over previous
import functools

import jax
import jax.numpy as jnp
from jax import lax
from jax.experimental import pallas as pl
from jax.experimental.pallas import tpu as pltpu
from jax.experimental.pallas import tpu_sc as plsc

MAXLEN = 200
EMBED = 32
BATCH = 16384

NC = 2
NS = 16
NW = NC * NS

SEQ_PER_W = BATCH // NW
C = 4
ROWS = C * MAXLEN
NCHUNK = SEQ_PER_W // C
GSUB = 100
NG = ROWS // GSUB
XROW_PER_W = (SEQ_PER_W * MAXLEN) // GSUB
ROW_PER_W = SEQ_PER_W * MAXLEN
NBUF = 4
NBLK = NCHUNK // NBUF

_mesh = plsc.VectorSubcoreMesh(core_axis_name="c", subcore_axis_name="s")


@functools.partial(
    pl.kernel,
    mesh=_mesh,
    compiler_params=pltpu.CompilerParams(use_tc_tiling_on_sc=False),
    out_type=jax.ShapeDtypeStruct((BATCH * MAXLEN, EMBED), jnp.float32),
    scratch_types=[
        pltpu.VMEM((NBUF, NG, GSUB), jnp.int32),
        pltpu.VMEM((NBUF, ROWS, EMBED), jnp.float32),
        pltpu.VMEM((MAXLEN, EMBED), jnp.float32),
        pltpu.SemaphoreType.DMA((NBUF,)),
        pltpu.SemaphoreType.DMA((NBUF,)),
    ],
)
def _embed_kernel(x_hbm, tok_hbm, pos_hbm, out_hbm, idx_v, buf, pos_v,
                  gsem, osem):
    wid = lax.axis_index("s") * NC + lax.axis_index("c")
    pltpu.sync_copy(pos_hbm, pos_v)

    def stage_and_fire(g, k):
        pltpu.sync_copy(
            x_hbm.at[pl.ds(wid * XROW_PER_W + g * NG, NG)], idx_v.at[k])
        for j in range(NG):
            pltpu.async_copy(
                tok_hbm.at[idx_v.at[k].at[j]],
                buf.at[k].at[pl.ds(j * GSUB, GSUB)],
                gsem.at[k],
            )

    def wait_gather(k):
        pltpu.make_async_copy(
            out_hbm.at[pl.ds(0, ROWS)], buf.at[k], gsem.at[k]).wait()

    def wait_out(k):
        pltpu.make_async_copy(
            buf.at[k], out_hbm.at[pl.ds(0, ROWS)], osem.at[k]).wait()

    stage_and_fire(0, 0)

    def block_body(b, carry):
        for k in range(NBUF):
            g = b * NBUF + k
            k1 = (k + 1) % NBUF
            gnext = g + 1

            @pl.when(jnp.logical_and(gnext < NCHUNK, gnext >= NBUF))
            def _():
                wait_out(k1)

            @pl.when(gnext < NCHUNK)
            def _():
                stage_and_fire(gnext, k1)

            wait_gather(k)

            def add_body(t, c2):
                p0 = pos_v[t, pl.ds(0, 16)]
                p1 = pos_v[t, pl.ds(16, 16)]
                for s in range(C):
                    r = s * MAXLEN + t
                    buf[k, r, pl.ds(0, 16)] += p0
                    buf[k, r, pl.ds(16, 16)] += p1
                return c2

            lax.fori_loop(0, MAXLEN, add_body, 0)
            pltpu.async_copy(
                buf.at[k],
                out_hbm.at[pl.ds(wid * ROW_PER_W + g * ROWS, ROWS)],
                osem.at[k],
            )
        return carry

    lax.fori_loop(0, NBLK, block_body, 0)
    for k in range(NBUF):
        wait_out(k)


def kernel(x, token_table, pos_table):
    x_flat = x.reshape(-1).astype(jnp.int32).reshape(-1, GSUB)
    out = _embed_kernel(x_flat, token_table, pos_table)
    return out.reshape(BATCH, MAXLEN, EMBED)
